# baseline (speedup 1.0000x reference)
import functools

import jax
import jax.numpy as jnp
from jax import lax
from jax.experimental import pallas as pl
from jax.experimental.pallas import tpu as pltpu
from jax.experimental.pallas import tpu_sc as plsc

_NUM_CORES = 2
_NUM_SUBCORES = 16
_NUM_WORKERS = _NUM_CORES * _NUM_SUBCORES
_T_CHUNK = 32
_S_CHUNK = 16
_TILE_RING = 3
_SP_RING = 2
_N_TILE = 5
_N_SP = 6


class _Pipe:
    def __init__(self, n, ring, load_fn, store_fn):
        self.n = n
        self.ring = ring
        self.load_fn = load_fn
        self.store_fn = store_fn
        self.loads = {}
        self.stores = {}
        self.store_waited = set()

    def prime(self):
        for k in range(min(self.ring, self.n)):
            self.loads[k] = self.load_fn(k, k % self.ring)

    def step(self, k):
        if k >= self.n:
            return
        self.loads[k].wait()
        self.stores[k] = self.store_fn(k, k % self.ring)
        j = k + self.ring
        if j < self.n:
            self.stores[k].wait()
            self.store_waited.add(k)
            self.loads[j] = self.load_fn(j, j % self.ring)

    def drain(self):
        for k in range(self.n):
            if k not in self.store_waited:
                self.stores[k].wait()


def _copy_body(rows_per_worker, w_hbm, out_hbm, shared, *scratch):
    sid = lax.axis_index("s")
    wid = sid * _NUM_CORES + lax.axis_index("c")
    base = wid * rows_per_worker
    sp_base = base + _N_TILE * _T_CHUNK
    tbufs = scratch[:_TILE_RING]
    tsin = scratch[_TILE_RING:2 * _TILE_RING]
    tsout = scratch[2 * _TILE_RING:3 * _TILE_RING]
    ssin = scratch[3 * _TILE_RING:3 * _TILE_RING + _SP_RING]
    ssout = scratch[3 * _TILE_RING + _SP_RING:3 * _TILE_RING + 2 * _SP_RING]

    tile_pipe = _Pipe(
        _N_TILE, _TILE_RING,
        lambda k, b: pltpu.async_copy(
            w_hbm.at[pl.ds(base + k * _T_CHUNK, _T_CHUNK)], tbufs[b], tsin[b]),
        lambda k, b: pltpu.async_copy(
            tbufs[b], out_hbm.at[pl.ds(base + k * _T_CHUNK, _T_CHUNK)],
            tsout[b]))
    sp_pipe = _Pipe(
        _N_SP, _SP_RING,
        lambda k, b: pltpu.async_copy(
            w_hbm.at[pl.ds(sp_base + k * _S_CHUNK, _S_CHUNK)],
            shared.at[sid, b], ssin[b]),
        lambda k, b: pltpu.async_copy(
            shared.at[sid, b],
            out_hbm.at[pl.ds(sp_base + k * _S_CHUNK, _S_CHUNK)], ssout[b]))

    tile_pipe.prime()
    sp_pipe.prime()
    for k in range(max(_N_TILE, _N_SP)):
        tile_pipe.step(k)
        sp_pipe.step(k)
    tile_pipe.drain()
    sp_pipe.drain()


def kernel(x, W):
    seq_len = x.shape[1]
    d_model = W.shape[1]
    rows_per_worker = _N_TILE * _T_CHUNK + _N_SP * _S_CHUNK
    assert seq_len == _NUM_WORKERS * rows_per_worker

    mesh = plsc.VectorSubcoreMesh(core_axis_name="c", subcore_axis_name="s")
    body = functools.partial(_copy_body, rows_per_worker)
    f = pl.kernel(
        body,
        mesh=mesh,
        out_type=jax.ShapeDtypeStruct((seq_len, d_model), W.dtype),
        scratch_types=(
            [pltpu.VMEM_SHARED((_NUM_SUBCORES, _SP_RING, _S_CHUNK, d_model),
                               W.dtype)]
            + [pltpu.VMEM((_T_CHUNK, d_model), W.dtype)
               for _ in range(_TILE_RING)]
            + [pltpu.SemaphoreType.DMA
               for _ in range(2 * _TILE_RING + 2 * _SP_RING)]
        ),
    )
    out = f(W)
    return out[None]


# speedup vs baseline: 1.0081x; 1.0081x over previous
import functools

import jax
import jax.numpy as jnp
from jax import lax
from jax.experimental import pallas as pl
from jax.experimental.pallas import tpu as pltpu
from jax.experimental.pallas import tpu_sc as plsc

_NUM_CORES = 2
_NUM_SUBCORES = 16
_NUM_WORKERS = _NUM_CORES * _NUM_SUBCORES
_CHUNK = 32
_RING = 3


def _copy_body(rows_per_worker, n_chunks, w_hbm, out_hbm, *scratch):
    wid = lax.axis_index("s") * _NUM_CORES + lax.axis_index("c")
    base = wid * rows_per_worker
    bufs = scratch[:_RING]
    sin = scratch[_RING:2 * _RING]
    sout = scratch[2 * _RING:3 * _RING]

    def load(i):
        return pltpu.async_copy(
            w_hbm.at[pl.ds(base + i * _CHUNK, _CHUNK)],
            bufs[i % _RING], sin[i % _RING])

    def store(i):
        return pltpu.async_copy(
            bufs[i % _RING],
            out_hbm.at[pl.ds(base + i * _CHUNK, _CHUNK)], sout[i % _RING])

    loads = {}
    for j in range(min(_RING, n_chunks)):
        loads[j] = load(j)
    stores = {}
    for i in range(n_chunks):
        loads[i].wait()
        stores[i] = store(i)
        j = i + _RING
        if j < n_chunks:
            stores[i].wait()
            loads[j] = load(j)
    for i in range(max(0, n_chunks - _RING), n_chunks):
        stores[i].wait()


def kernel(x, W):
    seq_len = x.shape[1]
    d_model = W.shape[1]
    assert seq_len % (_NUM_WORKERS * _CHUNK) == 0
    rows_per_worker = seq_len // _NUM_WORKERS
    n_chunks = rows_per_worker // _CHUNK

    mesh = plsc.VectorSubcoreMesh(core_axis_name="c", subcore_axis_name="s")
    body = functools.partial(_copy_body, rows_per_worker, n_chunks)
    f = pl.kernel(
        body,
        mesh=mesh,
        out_type=jax.ShapeDtypeStruct((seq_len, d_model), W.dtype),
        scratch_types=(
            [pltpu.VMEM((_CHUNK, d_model), W.dtype) for _ in range(_RING)]
            + [pltpu.SemaphoreType.DMA for _ in range(2 * _RING)]
        ),
    )
    out = f(W)
    return out[None]
